# Initial kernel scaffold; baseline (speedup 1.0000x reference)
import jax
import jax.numpy as jnp
from jax.experimental import pallas as pl


def kernel(graphs, spec_type, normal_type):
    raise NotImplementedError("write your pallas kernel here")


# speedup vs baseline: 1.3196x; 1.3196x over previous
import functools

import jax
import jax.numpy as jnp
from jax import lax
from jax.experimental import pallas as pl
from jax.experimental.pallas import tpu as pltpu
from jax.experimental.pallas import tpu_sc as plsc

_NC, _NS = 2, 16
_NW = _NC * _NS
_BATCH, _N, _D = 4, 256, 512
_B = _BATCH * _N * _N
_BPW = _B // _NW
_CHUNK = 64
_NCHUNK = _BPW // _CHUNK


def _sym_add_body(x_ref, o_ref):
    x = x_ref[0]
    o_ref[0] = x + x.T


@jax.jit
def _sym_add(graphs):
    return pl.pallas_call(
        _sym_add_body,
        grid=(_BATCH,),
        in_specs=[pl.BlockSpec((1, _N, _N), lambda b: (b, 0, 0))],
        out_specs=pl.BlockSpec((1, _N, _N), lambda b: (b, 0, 0)),
        out_shape=jax.ShapeDtypeStruct((_BATCH, _N, _N), jnp.int32),
    )(graphs)


_sc_mesh = plsc.VectorSubcoreMesh(
    core_axis_name="c", subcore_axis_name="s", num_cores=_NC, num_subcores=_NS
)


@functools.partial(
    pl.kernel,
    out_type=jax.ShapeDtypeStruct((_B, _D), jnp.float32),
    mesh=_sc_mesh,
    scratch_types=[
        pltpu.VMEM((_BPW,), jnp.int32),
        pltpu.VMEM((_CHUNK, _D), jnp.float32),
        pltpu.VMEM((_CHUNK, _D), jnp.float32),
        pltpu.SemaphoreType.DMA,
        pltpu.SemaphoreType.DMA,
    ],
)
def _sc_gather(table_hbm, idx_hbm, out_hbm, idx_v, rows0, rows1, sem0, sem1):
    wid = lax.axis_index("s") * _NC + lax.axis_index("c")
    base = wid * _BPW
    pltpu.sync_copy(idx_hbm.at[pl.ds(base, _BPW)], idx_v)

    rows = (rows0, rows1)
    sems = (sem0, sem1)

    def gather_start(c, buf):
        pltpu.async_copy(
            table_hbm.at[idx_v.at[pl.ds(c * _CHUNK, _CHUNK)]], rows[buf], sems[buf]
        )

    def gather_wait(buf):
        pltpu.make_async_copy(table_hbm.at[idx_v.at[pl.ds(0, _CHUNK)]],
                              rows[buf], sems[buf]).wait()

    gather_start(0, 0)
    gather_start(1, 1)

    def step(i, carry):
        for b in range(2):
            c = 2 * i + b
            gather_wait(b)
            pltpu.sync_copy(rows[b], out_hbm.at[pl.ds(base + c * _CHUNK, _CHUNK)])

            @pl.when(c + 2 < _NCHUNK)
            def _restart():
                gather_start(c + 2, b)

        return carry

    lax.fori_loop(0, _NCHUNK // 2, step, None)


def kernel(graphs, spec_type, normal_type):
    table = jnp.concatenate((spec_type, normal_type), axis=0)
    g = _sym_add(graphs)
    flat = _sc_gather(table, g.reshape(_B))
    return flat.reshape(_BATCH, _N, _N, _D)


# speedup vs baseline: 1.3337x; 1.0107x over previous
import functools

import jax
import jax.numpy as jnp
from jax import lax
from jax.experimental import pallas as pl
from jax.experimental.pallas import tpu as pltpu
from jax.experimental.pallas import tpu_sc as plsc

_NC, _NS = 2, 16
_NW = _NC * _NS
_BATCH, _N, _D = 4, 256, 512
_B = _BATCH * _N * _N
_BPW = _B // _NW
_CHUNK = 32
_NBUF = 4
_NCHUNK = _BPW // _CHUNK


def _sym_add_body(x_ref, o_ref):
    x = x_ref[0]
    o_ref[0] = x + x.T


@jax.jit
def _sym_add(graphs):
    return pl.pallas_call(
        _sym_add_body,
        grid=(_BATCH,),
        in_specs=[pl.BlockSpec((1, _N, _N), lambda b: (b, 0, 0))],
        out_specs=pl.BlockSpec((1, _N, _N), lambda b: (b, 0, 0)),
        out_shape=jax.ShapeDtypeStruct((_BATCH, _N, _N), jnp.int32),
    )(graphs)


_sc_mesh = plsc.VectorSubcoreMesh(
    core_axis_name="c", subcore_axis_name="s", num_cores=_NC, num_subcores=_NS
)


@functools.partial(
    pl.kernel,
    out_type=jax.ShapeDtypeStruct((_B, _D), jnp.float32),
    mesh=_sc_mesh,
    scratch_types=[
        pltpu.VMEM((_BPW,), jnp.int32),
        pltpu.VMEM((_NBUF, _CHUNK, _D), jnp.float32),
        [pltpu.SemaphoreType.DMA] * _NBUF,
        [pltpu.SemaphoreType.DMA] * _NBUF,
    ],
)
def _sc_gather(table_hbm, idx_hbm, out_hbm, idx_v, rows_v, gsems, ssems):
    wid = lax.axis_index("s") * _NC + lax.axis_index("c")
    base = wid * _BPW
    pltpu.sync_copy(idx_hbm.at[pl.ds(base, _BPW)], idx_v)

    def gather_start(c, buf):
        pltpu.async_copy(
            table_hbm.at[idx_v.at[pl.ds(c * _CHUNK, _CHUNK)]],
            rows_v.at[buf], gsems[buf],
        )

    def gather_wait(buf):
        pltpu.make_async_copy(
            table_hbm.at[idx_v.at[pl.ds(0, _CHUNK)]], rows_v.at[buf], gsems[buf]
        ).wait()

    def scatter_start(c, buf):
        pltpu.async_copy(
            rows_v.at[buf], out_hbm.at[pl.ds(base + c * _CHUNK, _CHUNK)], ssems[buf]
        )

    def scatter_wait(buf):
        pltpu.make_async_copy(
            rows_v.at[buf], out_hbm.at[pl.ds(0, _CHUNK)], ssems[buf]
        ).wait()

    gather_start(0, 0)
    gather_start(1, 1)

    def step(i, carry):
        for b in range(_NBUF):
            c = _NBUF * i + b
            b2 = (b + 2) % _NBUF

            @pl.when(c >= 2)
            def _drain():
                scatter_wait(b2)

            @pl.when(c + 2 < _NCHUNK)
            def _refill():
                gather_start(c + 2, b2)

            gather_wait(b)
            scatter_start(c, b)
        return carry

    lax.fori_loop(0, _NCHUNK // _NBUF, step, None)
    scatter_wait((_NCHUNK - 2) % _NBUF)
    scatter_wait((_NCHUNK - 1) % _NBUF)


def kernel(graphs, spec_type, normal_type):
    table = jnp.concatenate((spec_type, normal_type), axis=0)
    g = _sym_add(graphs)
    flat = _sc_gather(table, g.reshape(_B))
    return flat.reshape(_BATCH, _N, _N, _D)
